# baseline (speedup 1.0000x reference)
import functools

import jax
import jax.numpy as jnp
from jax import lax
from jax.experimental import pallas as pl
from jax.experimental.pallas import tpu as pltpu
from jax.experimental.pallas import tpu_sc as plsc

N_NODES = 10000
N_EDGES = 160000
D_IN = 384
D_HID = 16
D_OUT = 8

NC = 2
NS = 16
CHUNK = 128
K = 40
E_PAD = NC * NS * K * CHUNK
N_PAD = 10240
ROWS_PER_TILE = N_PAD // NS
TROWS = N_NODES // NS

_mesh = plsc.VectorSubcoreMesh(
    core_axis_name="c", subcore_axis_name="s", num_cores=NC, num_subcores=NS)
_sc_params = pltpu.CompilerParams(use_tc_tiling_on_sc=False)


@functools.partial(
    pl.kernel,
    out_type=jax.ShapeDtypeStruct((NC, N_PAD, D_HID), jnp.float32),
    mesh=_mesh,
    scratch_types=[
        pltpu.VMEM((K, CHUNK), jnp.int32),
        pltpu.VMEM((CHUNK, D_HID), jnp.float32),
        pltpu.VMEM_SHARED((N_PAD, D_HID), jnp.float32),
        pltpu.SemaphoreType.DMA,
        pltpu.SemaphoreType.DMA,
        pltpu.SemaphoreType.DMA,
        pltpu.SemaphoreType.DMA,
    ],
    compiler_params=_sc_params,
)
def _deg_kernel(dst_hbm, ones_hbm, zeros_hbm, out_hbm, dst_v, ones_v, acc,
                dsem0, dsem1, dsem2, dsem3):
    c = lax.axis_index("c")
    s = lax.axis_index("s")
    pltpu.sync_copy(dst_hbm.at[c, s], dst_v)
    pltpu.sync_copy(ones_hbm, ones_v)
    pltpu.sync_copy(zeros_hbm, acc.at[pl.ds(s * ROWS_PER_TILE, ROWS_PER_TILE)])
    plsc.subcore_barrier()

    dsems = (dsem0, dsem1, dsem2, dsem3)

    def body(i, carry):
        cps = [pltpu.async_copy(ones_v, acc.at[dst_v.at[4 * i + u]],
                                dsems[u], add=True) for u in range(4)]
        for cp in cps:
            cp.wait()
        return carry

    lax.fori_loop(0, K // 4, body, 0)
    plsc.subcore_barrier()
    sl = pl.ds(s * ROWS_PER_TILE, ROWS_PER_TILE)
    pltpu.sync_copy(acc.at[sl], out_hbm.at[c].at[sl])


@functools.partial(
    pl.kernel,
    out_type=jax.ShapeDtypeStruct((NC, N_PAD, D_HID), jnp.float32),
    mesh=_mesh,
    scratch_types=[
        pltpu.VMEM((K, CHUNK), jnp.int32),
        pltpu.VMEM((K, CHUNK), jnp.int32),
        pltpu.VMEM((4, CHUNK, D_HID), jnp.float32),
        pltpu.VMEM_SHARED((N_PAD, D_HID), jnp.float32),
        pltpu.VMEM_SHARED((N_PAD, D_HID), jnp.float32),
        [pltpu.SemaphoreType.DMA] * 4,
        [pltpu.SemaphoreType.DMA] * 4,
    ],
    compiler_params=_sc_params,
)
def _agg_kernel(src_hbm, dst_hbm, ys_hbm, zeros_hbm, out_hbm,
                src_v, dst_v, rows_v, table, acc, gsems, ssems):
    c = lax.axis_index("c")
    s = lax.axis_index("s")
    pltpu.sync_copy(src_hbm.at[c, s], src_v)
    pltpu.sync_copy(dst_hbm.at[c, s], dst_v)
    tsl = pl.ds(s * TROWS, TROWS)
    pltpu.sync_copy(ys_hbm.at[tsl], table.at[tsl])
    pltpu.sync_copy(zeros_hbm, acc.at[pl.ds(s * ROWS_PER_TILE, ROWS_PER_TILE)])
    plsc.subcore_barrier()

    def gath(j, u):
        return pltpu.async_copy(table.at[src_v.at[j]], rows_v.at[u], gsems[u])

    def scat(j, u):
        return pltpu.async_copy(rows_v.at[u], acc.at[dst_v.at[j]], ssems[u],
                                add=True)

    def body(i, carry):
        j = 4 * i
        g0 = gath(j, 0)
        g1 = gath(j + 1, 1)
        g0.wait()
        s0 = scat(j, 0)
        g1.wait()
        s1 = scat(j + 1, 1)
        g2 = gath(j + 2, 2)
        g3 = gath(j + 3, 3)
        g2.wait()
        s2 = scat(j + 2, 2)
        g3.wait()
        s3 = scat(j + 3, 3)
        s0.wait()
        s1.wait()
        s2.wait()
        s3.wait()
        return carry

    lax.fori_loop(0, K // 4, body, 0)
    plsc.subcore_barrier()
    sl = pl.ds(s * ROWS_PER_TILE, ROWS_PER_TILE)
    pltpu.sync_copy(acc.at[sl], out_hbm.at[c].at[sl])


MROWS = N_PAD * D_HID // 128
_BR = 256
_GRID = MROWS // _BR


def _tc1_body(x_ref, w1blk_ref, degp_ref, ys_ref, dis_ref):
    deg = degp_ref[0] + degp_ref[1] + 1.0
    dis = lax.rsqrt(deg)
    xw = jnp.dot(x_ref[...], w1blk_ref[...],
                 preferred_element_type=jnp.float32)
    dis_ref[...] = dis
    ys_ref[...] = dis * xw


def _tc2_body(p_ref, dis_ref, ys_ref, b1_ref, w2blk_ref, t2_ref):
    dis = dis_ref[...]
    pre = dis * (p_ref[0] + p_ref[1]) + dis * ys_ref[...] + b1_ref[...]
    h = jnp.maximum(pre, 0.0)
    hw2 = jnp.dot(h, w2blk_ref[...], preferred_element_type=jnp.float32)
    t2_ref[...] = dis * hw2


def _tc3_body(q_ref, dis_ref, t2_ref, b2_ref, ssum_ref, sbc_ref, out_ref):
    dis = dis_ref[...]
    t = dis * (q_ref[0] + q_ref[1]) + dis * t2_ref[...] + b2_ref[...]
    e = jnp.exp(t)
    s8 = jnp.dot(e, ssum_ref[...], preferred_element_type=jnp.float32,
                 precision=lax.Precision.HIGHEST)
    lse = jnp.log(s8)
    lseb = jnp.dot(lse, sbc_ref[...], preferred_element_type=jnp.float32,
                   precision=lax.Precision.HIGHEST)
    out_ref[...] = t - lseb


_tc1 = pl.pallas_call(
    _tc1_body,
    grid=(_GRID,),
    in_specs=[
        pl.BlockSpec((_BR, D_IN * 8), lambda i: (i, 0)),
        pl.BlockSpec((D_IN * 8, 128), lambda i: (0, 0)),
        pl.BlockSpec((NC, _BR, 128), lambda i: (0, i, 0)),
    ],
    out_specs=[
        pl.BlockSpec((_BR, 128), lambda i: (i, 0)),
        pl.BlockSpec((_BR, 128), lambda i: (i, 0)),
    ],
    out_shape=[
        jax.ShapeDtypeStruct((MROWS, 128), jnp.float32),
        jax.ShapeDtypeStruct((MROWS, 128), jnp.float32),
    ],
)

_tc2 = pl.pallas_call(
    _tc2_body,
    grid=(1,),
    in_specs=[
        pl.BlockSpec((NC, MROWS, 128), lambda i: (0, 0, 0)),
        pl.BlockSpec((MROWS, 128), lambda i: (0, 0)),
        pl.BlockSpec((MROWS, 128), lambda i: (0, 0)),
        pl.BlockSpec((128,), lambda i: (0,)),
        pl.BlockSpec((128, 128), lambda i: (0, 0)),
    ],
    out_specs=[pl.BlockSpec((MROWS, 128), lambda i: (0, 0))],
    out_shape=[jax.ShapeDtypeStruct((MROWS, 128), jnp.float32)],
)

_tc3 = pl.pallas_call(
    _tc3_body,
    grid=(1,),
    in_specs=[
        pl.BlockSpec((NC, MROWS, 128), lambda i: (0, 0, 0)),
        pl.BlockSpec((MROWS, 128), lambda i: (0, 0)),
        pl.BlockSpec((MROWS, 128), lambda i: (0, 0)),
        pl.BlockSpec((128,), lambda i: (0,)),
        pl.BlockSpec((128, 8), lambda i: (0, 0)),
        pl.BlockSpec((8, 128), lambda i: (0, 0)),
    ],
    out_specs=[pl.BlockSpec((MROWS, 128), lambda i: (0, 0))],
    out_shape=[jax.ShapeDtypeStruct((MROWS, 128), jnp.float32)],
)


def kernel(x, edge_index, W1, b1, W2, b2):
    npad = E_PAD - N_EDGES
    padidx = N_NODES + jnp.arange(npad, dtype=jnp.int32) % (N_PAD - N_NODES)
    srcp = jnp.concatenate([edge_index[0].astype(jnp.int32), padidx])
    dstp = jnp.concatenate([edge_index[1].astype(jnp.int32), padidx])
    srcp = srcp.reshape(NC, NS, K, CHUNK)
    dstp = dstp.reshape(NC, NS, K, CHUNK)

    ones16 = jnp.ones((CHUNK, D_HID), jnp.float32)
    zeros16 = jnp.zeros((ROWS_PER_TILE, D_HID), jnp.float32)
    eye8 = jnp.eye(8, dtype=jnp.float32)
    w1blk = jnp.kron(eye8, W1).astype(jnp.bfloat16)
    w2pad = jnp.pad(W2, ((0, 0), (0, D_HID - D_OUT)))
    w2blk = jnp.kron(eye8, w2pad)
    b1t = jnp.tile(b1, 8)
    b2t = jnp.tile(jnp.pad(b2, (0, D_HID - D_OUT)), 8)
    lane = jnp.arange(128, dtype=jnp.int32)
    ssum = ((lane[:, None] // D_HID == jnp.arange(8)[None, :])
            & (lane[:, None] % D_HID < D_OUT)).astype(jnp.float32)
    sbc = (lane[None, :] // D_HID == jnp.arange(8)[:, None]).astype(
        jnp.float32)
    x2 = x.astype(jnp.bfloat16).reshape(N_NODES // 8, D_IN * 8)

    degp = _deg_kernel(dstp, ones16, zeros16)
    ys2d, dis2d = _tc1(x2, w1blk, degp.reshape(NC, MROWS, 128))
    p1 = _agg_kernel(srcp, dstp, ys2d.reshape(N_PAD, D_HID), zeros16)
    (t2,) = _tc2(p1.reshape(NC, MROWS, 128), dis2d, ys2d, b1t, w2blk)
    p2 = _agg_kernel(srcp, dstp, t2.reshape(N_PAD, D_HID), zeros16)
    (outp,) = _tc3(p2.reshape(NC, MROWS, 128), dis2d, t2, b2t, ssum, sbc)
    return outp.reshape(N_PAD, D_HID)[:N_NODES, :D_OUT]


# speedup vs baseline: 1.1025x; 1.1025x over previous
import functools

import jax
import jax.numpy as jnp
from jax import lax
from jax.experimental import pallas as pl
from jax.experimental.pallas import tpu as pltpu
from jax.experimental.pallas import tpu_sc as plsc

N_NODES = 10000
N_EDGES = 160000
D_IN = 384
D_HID = 16
D_OUT = 8

NC = 2
NS = 16
CHUNK = 128
K = 40
E_PAD = NC * NS * K * CHUNK
N_PAD = 10240
ROWS_PER_TILE = N_PAD // NS
TROWS = N_NODES // NS

_mesh = plsc.VectorSubcoreMesh(
    core_axis_name="c", subcore_axis_name="s", num_cores=NC, num_subcores=NS)
_sc_params = pltpu.CompilerParams(use_tc_tiling_on_sc=False)


@functools.partial(
    pl.kernel,
    out_type=jax.ShapeDtypeStruct((NC, N_PAD, D_HID), jnp.float32),
    mesh=_mesh,
    scratch_types=[
        pltpu.VMEM((K, CHUNK), jnp.int32),
        pltpu.VMEM((CHUNK, D_HID), jnp.float32),
        pltpu.VMEM_SHARED((N_PAD, D_HID), jnp.float32),
        pltpu.SemaphoreType.DMA,
        pltpu.SemaphoreType.DMA,
        pltpu.SemaphoreType.DMA,
        pltpu.SemaphoreType.DMA,
    ],
    compiler_params=_sc_params,
)
def _deg_kernel(dst_hbm, ones_hbm, zeros_hbm, out_hbm, dst_v, ones_v, acc,
                dsem0, dsem1, dsem2, dsem3):
    c = lax.axis_index("c")
    s = lax.axis_index("s")
    pltpu.sync_copy(dst_hbm.at[c, s], dst_v)
    pltpu.sync_copy(ones_hbm, ones_v)
    pltpu.sync_copy(zeros_hbm, acc.at[pl.ds(s * ROWS_PER_TILE, ROWS_PER_TILE)])
    plsc.subcore_barrier()

    dsems = (dsem0, dsem1, dsem2, dsem3)

    def body(i, carry):
        cps = [pltpu.async_copy(ones_v, acc.at[dst_v.at[4 * i + u]],
                                dsems[u], add=True) for u in range(4)]
        for cp in cps:
            cp.wait()
        return carry

    lax.fori_loop(0, K // 4, body, 0)
    plsc.subcore_barrier()
    sl = pl.ds(s * ROWS_PER_TILE, ROWS_PER_TILE)
    pltpu.sync_copy(acc.at[sl], out_hbm.at[c].at[sl])


@functools.partial(
    pl.kernel,
    out_type=jax.ShapeDtypeStruct((NC, N_PAD, D_HID), jnp.float32),
    mesh=_mesh,
    scratch_types=[
        pltpu.VMEM((K, CHUNK), jnp.int32),
        pltpu.VMEM((K, CHUNK), jnp.int32),
        pltpu.VMEM((4, CHUNK, D_HID), jnp.float32),
        pltpu.VMEM_SHARED((N_PAD, D_HID), jnp.float32),
        pltpu.VMEM_SHARED((N_PAD, D_HID), jnp.float32),
        [pltpu.SemaphoreType.DMA] * 4,
        [pltpu.SemaphoreType.DMA] * 4,
    ],
    compiler_params=_sc_params,
)
def _agg_kernel(src_hbm, dst_hbm, ys_hbm, zeros_hbm, out_hbm,
                src_v, dst_v, rows_v, table, acc, gsems, ssems):
    c = lax.axis_index("c")
    s = lax.axis_index("s")
    pltpu.sync_copy(src_hbm.at[c, s], src_v)
    pltpu.sync_copy(dst_hbm.at[c, s], dst_v)
    tsl = pl.ds(s * TROWS, TROWS)
    pltpu.sync_copy(ys_hbm.at[tsl], table.at[tsl])
    pltpu.sync_copy(zeros_hbm, acc.at[pl.ds(s * ROWS_PER_TILE, ROWS_PER_TILE)])
    plsc.subcore_barrier()

    def gath(j, u):
        return pltpu.async_copy(table.at[src_v.at[j]], rows_v.at[u], gsems[u])

    def scat(j, u):
        return pltpu.async_copy(rows_v.at[u], acc.at[dst_v.at[j]], ssems[u],
                                add=True)

    def body(i, carry):
        j = 4 * i
        g0 = gath(j, 0)
        g1 = gath(j + 1, 1)
        g0.wait()
        s0 = scat(j, 0)
        g1.wait()
        s1 = scat(j + 1, 1)
        g2 = gath(j + 2, 2)
        g3 = gath(j + 3, 3)
        g2.wait()
        s2 = scat(j + 2, 2)
        g3.wait()
        s3 = scat(j + 3, 3)
        s0.wait()
        s1.wait()
        s2.wait()
        s3.wait()
        return carry

    lax.fori_loop(0, K // 4, body, 0)
    plsc.subcore_barrier()
    sl = pl.ds(s * ROWS_PER_TILE, ROWS_PER_TILE)
    pltpu.sync_copy(acc.at[sl], out_hbm.at[c].at[sl])


MROWS = N_PAD * D_HID // 128
_BR = 256
_GRID = MROWS // _BR


def _tc1_body(x_ref, w1blk_ref, degp_ref, ys_ref, dis_ref):
    deg = degp_ref[0] + degp_ref[1] + 1.0
    dis = lax.rsqrt(deg)
    xb = x_ref[...].astype(jnp.bfloat16)
    xw = jnp.dot(xb, w1blk_ref[...], preferred_element_type=jnp.float32)
    dis_ref[...] = dis
    ys_ref[...] = dis * xw


def _tc2_body(p_ref, dis_ref, ys_ref, b1_ref, w2blk_ref, t2_ref):
    dis = dis_ref[...]
    pre = dis * (p_ref[0] + p_ref[1]) + dis * ys_ref[...] + b1_ref[...]
    h = jnp.maximum(pre, 0.0)
    hw2 = jnp.dot(h, w2blk_ref[...], preferred_element_type=jnp.float32)
    t2_ref[...] = dis * hw2


def _tc3_body(q_ref, dis_ref, t2_ref, b2_ref, ssum_ref, sbc_ref, out_ref):
    dis = dis_ref[...]
    t = dis * (q_ref[0] + q_ref[1]) + dis * t2_ref[...] + b2_ref[...]
    e = jnp.exp(t)
    s8 = jnp.dot(e, ssum_ref[...], preferred_element_type=jnp.float32,
                 precision=lax.Precision.HIGHEST)
    lse = jnp.log(s8)
    lseb = jnp.dot(lse, sbc_ref[...], preferred_element_type=jnp.float32,
                   precision=lax.Precision.HIGHEST)
    out_ref[...] = t - lseb


_tc1 = pl.pallas_call(
    _tc1_body,
    grid=(_GRID,),
    in_specs=[
        pl.BlockSpec((_BR, D_IN * 8), lambda i: (i, 0)),
        pl.BlockSpec((D_IN * 8, 128), lambda i: (0, 0)),
        pl.BlockSpec((NC, _BR, 128), lambda i: (0, i, 0)),
    ],
    out_specs=[
        pl.BlockSpec((_BR, 128), lambda i: (i, 0)),
        pl.BlockSpec((_BR, 128), lambda i: (i, 0)),
    ],
    out_shape=[
        jax.ShapeDtypeStruct((MROWS, 128), jnp.float32),
        jax.ShapeDtypeStruct((MROWS, 128), jnp.float32),
    ],
)

_tc2 = pl.pallas_call(
    _tc2_body,
    grid=(1,),
    in_specs=[
        pl.BlockSpec((NC, MROWS, 128), lambda i: (0, 0, 0)),
        pl.BlockSpec((MROWS, 128), lambda i: (0, 0)),
        pl.BlockSpec((MROWS, 128), lambda i: (0, 0)),
        pl.BlockSpec((128,), lambda i: (0,)),
        pl.BlockSpec((128, 128), lambda i: (0, 0)),
    ],
    out_specs=[pl.BlockSpec((MROWS, 128), lambda i: (0, 0))],
    out_shape=[jax.ShapeDtypeStruct((MROWS, 128), jnp.float32)],
)

_tc3 = pl.pallas_call(
    _tc3_body,
    grid=(1,),
    in_specs=[
        pl.BlockSpec((NC, MROWS, 128), lambda i: (0, 0, 0)),
        pl.BlockSpec((MROWS, 128), lambda i: (0, 0)),
        pl.BlockSpec((MROWS, 128), lambda i: (0, 0)),
        pl.BlockSpec((128,), lambda i: (0,)),
        pl.BlockSpec((128, 8), lambda i: (0, 0)),
        pl.BlockSpec((8, 128), lambda i: (0, 0)),
    ],
    out_specs=[pl.BlockSpec((MROWS, 128), lambda i: (0, 0))],
    out_shape=[jax.ShapeDtypeStruct((MROWS, 128), jnp.float32)],
)


def kernel(x, edge_index, W1, b1, W2, b2):
    x2 = x.reshape(N_NODES // 8, D_IN * 8)
    npad = E_PAD - N_EDGES
    spread = N_NODES + jnp.arange(npad, dtype=jnp.int32) % (N_PAD - N_NODES)
    padconst = jnp.concatenate(
        [jnp.zeros((2, N_EDGES), jnp.int32),
         jnp.broadcast_to(spread, (2, npad))], axis=1)
    ep = jnp.pad(edge_index.astype(jnp.int32),
                 ((0, 0), (0, npad))) + padconst
    srcp = ep[0].reshape(NC, NS, K, CHUNK)
    dstp = ep[1].reshape(NC, NS, K, CHUNK)

    ones16 = jnp.ones((CHUNK, D_HID), jnp.float32)
    zeros16 = jnp.zeros((ROWS_PER_TILE, D_HID), jnp.float32)
    eye8 = jnp.eye(8, dtype=jnp.float32)
    w1blk = jnp.kron(eye8, W1).astype(jnp.bfloat16)
    w2pad = jnp.pad(W2, ((0, 0), (0, D_HID - D_OUT)))
    w2blk = jnp.kron(eye8, w2pad)
    b1t = jnp.tile(b1, 8)
    b2t = jnp.tile(jnp.pad(b2, (0, D_HID - D_OUT)), 8)
    lane = jnp.arange(128, dtype=jnp.int32)
    ssum = ((lane[:, None] // D_HID == jnp.arange(8)[None, :])
            & (lane[:, None] % D_HID < D_OUT)).astype(jnp.float32)
    sbc = (lane[None, :] // D_HID == jnp.arange(8)[:, None]).astype(
        jnp.float32)
    degp = _deg_kernel(dstp, ones16, zeros16)
    ys2d, dis2d = _tc1(x2, w1blk, degp.reshape(NC, MROWS, 128))
    p1 = _agg_kernel(srcp, dstp, ys2d.reshape(N_PAD, D_HID), zeros16)
    (t2,) = _tc2(p1.reshape(NC, MROWS, 128), dis2d, ys2d, b1t, w2blk)
    p2 = _agg_kernel(srcp, dstp, t2.reshape(N_PAD, D_HID), zeros16)
    (outp,) = _tc3(p2.reshape(NC, MROWS, 128), dis2d, t2, b2t, ssum, sbc)
    return outp.reshape(N_PAD, D_HID)[:N_NODES, :D_OUT]


# speedup vs baseline: 1.2246x; 1.1107x over previous
import functools

import jax
import jax.numpy as jnp
from jax import lax
from jax.experimental import pallas as pl
from jax.experimental.pallas import tpu as pltpu
from jax.experimental.pallas import tpu_sc as plsc

N_NODES = 10000
N_EDGES = 160000
D_IN = 384
D_HID = 16
D_OUT = 8

NC = 2
NS = 16
CHUNK = 128
K = 40
E_PAD = NC * NS * K * CHUNK
N_PAD = 10240
ROWS_PER_TILE = N_PAD // NS
TROWS = N_PAD // NS

_mesh = plsc.VectorSubcoreMesh(
    core_axis_name="c", subcore_axis_name="s", num_cores=NC, num_subcores=NS)
_sc_params = pltpu.CompilerParams(use_tc_tiling_on_sc=False)


@functools.partial(
    pl.kernel,
    out_type=jax.ShapeDtypeStruct((NC, N_PAD, D_HID), jnp.float32),
    mesh=_mesh,
    scratch_types=[
        pltpu.VMEM((K, CHUNK), jnp.int32),
        pltpu.VMEM((CHUNK, D_HID), jnp.float32),
        pltpu.VMEM_SHARED((N_PAD, D_HID), jnp.float32),
        pltpu.SemaphoreType.DMA,
        pltpu.SemaphoreType.DMA,
        pltpu.SemaphoreType.DMA,
        pltpu.SemaphoreType.DMA,
    ],
    compiler_params=_sc_params,
)
def _deg_kernel(dst_hbm, ones_hbm, zeros_hbm, out_hbm, dst_v, ones_v, acc,
                dsem0, dsem1, dsem2, dsem3):
    c = lax.axis_index("c")
    s = lax.axis_index("s")
    pltpu.sync_copy(dst_hbm.at[c, s], dst_v)
    pltpu.sync_copy(ones_hbm, ones_v)
    pltpu.sync_copy(zeros_hbm, acc.at[pl.ds(s * ROWS_PER_TILE, ROWS_PER_TILE)])
    plsc.subcore_barrier()

    dsems = (dsem0, dsem1, dsem2, dsem3)

    def body(i, carry):
        cps = [pltpu.async_copy(ones_v, acc.at[dst_v.at[4 * i + u]],
                                dsems[u], add=True) for u in range(4)]
        for cp in cps:
            cp.wait()
        return carry

    lax.fori_loop(0, K // 4, body, 0)
    plsc.subcore_barrier()
    sl = pl.ds(s * ROWS_PER_TILE, ROWS_PER_TILE)
    pltpu.sync_copy(acc.at[sl], out_hbm.at[c].at[sl])


@functools.partial(
    pl.kernel,
    out_type=jax.ShapeDtypeStruct((NC, N_PAD, D_HID), jnp.float32),
    mesh=_mesh,
    scratch_types=[
        pltpu.VMEM((K, CHUNK), jnp.int32),
        pltpu.VMEM((K, CHUNK), jnp.int32),
        pltpu.VMEM((4, CHUNK, D_HID), jnp.float32),
        pltpu.VMEM_SHARED((N_PAD, D_HID), jnp.float32),
        pltpu.VMEM_SHARED((N_PAD, D_HID), jnp.float32),
        [pltpu.SemaphoreType.DMA] * 4,
        [pltpu.SemaphoreType.DMA] * 4,
    ],
    compiler_params=_sc_params,
)
def _agg_kernel(src_hbm, dst_hbm, ys_hbm, zeros_hbm, out_hbm,
                src_v, dst_v, rows_v, table, acc, gsems, ssems):
    c = lax.axis_index("c")
    s = lax.axis_index("s")
    pltpu.sync_copy(src_hbm.at[c, s], src_v)
    pltpu.sync_copy(dst_hbm.at[c, s], dst_v)
    tsl = pl.ds(s * TROWS, TROWS)
    pltpu.sync_copy(ys_hbm.at[tsl], table.at[tsl])
    pltpu.sync_copy(zeros_hbm, acc.at[pl.ds(s * ROWS_PER_TILE, ROWS_PER_TILE)])
    plsc.subcore_barrier()

    def gath(j, u):
        return pltpu.async_copy(table.at[src_v.at[j]], rows_v.at[u], gsems[u])

    def scat(j, u):
        return pltpu.async_copy(rows_v.at[u], acc.at[dst_v.at[j]], ssems[u],
                                add=True)

    def body(i, carry):
        j = 4 * i
        g0 = gath(j, 0)
        g1 = gath(j + 1, 1)
        g0.wait()
        s0 = scat(j, 0)
        g1.wait()
        s1 = scat(j + 1, 1)
        g2 = gath(j + 2, 2)
        g3 = gath(j + 3, 3)
        g2.wait()
        s2 = scat(j + 2, 2)
        g3.wait()
        s3 = scat(j + 3, 3)
        s0.wait()
        s1.wait()
        s2.wait()
        s3.wait()
        return carry

    lax.fori_loop(0, K // 4, body, 0)
    plsc.subcore_barrier()
    sl = pl.ds(s * ROWS_PER_TILE, ROWS_PER_TILE)
    pltpu.sync_copy(acc.at[sl], out_hbm.at[c].at[sl])


MROWS = N_PAD * D_HID // 128
_BR = 256
_GRID = MROWS // _BR


def _tc1_body(x0, x1, x2, x3, x4, x5, x6, x7, w1_ref, degp_ref,
              ys_ref, dis_ref):
    deg = degp_ref[0] + degp_ref[1] + 1.0
    dis = lax.rsqrt(deg)
    w1 = w1_ref[...]
    xw = jnp.concatenate(
        [jnp.dot(xr[...].astype(jnp.bfloat16), w1,
                 preferred_element_type=jnp.float32)
         for xr in (x0, x1, x2, x3, x4, x5, x6, x7)], axis=1)
    dis_ref[...] = dis
    ys_ref[...] = dis * xw


def _tc2_body(p_ref, dis_ref, ys_ref, b1_ref, w2blk_ref, t2_ref):
    dis = dis_ref[...]
    pre = dis * (p_ref[0] + p_ref[1]) + dis * ys_ref[...] + b1_ref[...]
    h = jnp.maximum(pre, 0.0)
    hw2 = jnp.dot(h, w2blk_ref[...], preferred_element_type=jnp.float32)
    t2_ref[...] = dis * hw2


def _tc3_body(q_ref, dis_ref, t2_ref, b2_ref, ssum_ref, sbc_ref, out_ref):
    dis = dis_ref[...]
    t = dis * (q_ref[0] + q_ref[1]) + dis * t2_ref[...] + b2_ref[...]
    e = jnp.exp(t)
    s8 = jnp.dot(e, ssum_ref[...], preferred_element_type=jnp.float32,
                 precision=lax.Precision.HIGHEST)
    lse = jnp.log(s8)
    lseb = jnp.dot(lse, sbc_ref[...], preferred_element_type=jnp.float32,
                   precision=lax.Precision.HIGHEST)
    out_ref[...] = t - lseb


def _xspec(g):
    return pl.BlockSpec((_BR, D_IN), lambda i, g=g: (_GRID * g + i, 0))


_tc1 = pl.pallas_call(
    _tc1_body,
    grid=(_GRID,),
    in_specs=[_xspec(g) for g in range(8)] + [
        pl.BlockSpec((D_IN, D_HID), lambda i: (0, 0)),
        pl.BlockSpec((NC, _BR, 128), lambda i: (0, i, 0)),
    ],
    out_specs=[
        pl.BlockSpec((_BR, 128), lambda i: (i, 0)),
        pl.BlockSpec((_BR, 128), lambda i: (i, 0)),
    ],
    out_shape=[
        jax.ShapeDtypeStruct((MROWS, 128), jnp.float32),
        jax.ShapeDtypeStruct((MROWS, 128), jnp.float32),
    ],
)

_tc2 = pl.pallas_call(
    _tc2_body,
    grid=(1,),
    in_specs=[
        pl.BlockSpec((NC, MROWS, 128), lambda i: (0, 0, 0)),
        pl.BlockSpec((MROWS, 128), lambda i: (0, 0)),
        pl.BlockSpec((MROWS, 128), lambda i: (0, 0)),
        pl.BlockSpec((128,), lambda i: (0,)),
        pl.BlockSpec((128, 128), lambda i: (0, 0)),
    ],
    out_specs=[pl.BlockSpec((MROWS, 128), lambda i: (0, 0))],
    out_shape=[jax.ShapeDtypeStruct((MROWS, 128), jnp.float32)],
)

_tc3 = pl.pallas_call(
    _tc3_body,
    grid=(1,),
    in_specs=[
        pl.BlockSpec((NC, MROWS, 128), lambda i: (0, 0, 0)),
        pl.BlockSpec((MROWS, 128), lambda i: (0, 0)),
        pl.BlockSpec((MROWS, 128), lambda i: (0, 0)),
        pl.BlockSpec((128,), lambda i: (0,)),
        pl.BlockSpec((128, 8), lambda i: (0, 0)),
        pl.BlockSpec((8, 128), lambda i: (0, 0)),
    ],
    out_specs=[pl.BlockSpec((MROWS, 128), lambda i: (0, 0))],
    out_shape=[jax.ShapeDtypeStruct((MROWS, 128), jnp.float32)],
)


def kernel(x, edge_index, W1, b1, W2, b2):
    npad = E_PAD - N_EDGES
    q = N_NODES // 8
    em = (edge_index.astype(jnp.int32) % 1280) * 8 \
        + edge_index.astype(jnp.int32) // 1280
    spread = 8 * (1040 + jnp.arange(npad, dtype=jnp.int32) % 240) + 7
    padconst = jnp.concatenate(
        [jnp.zeros((2, N_EDGES), jnp.int32),
         jnp.broadcast_to(spread, (2, npad))], axis=1)
    ep = jnp.pad(em, ((0, 0), (0, npad))) + padconst
    srcp = ep[0].reshape(NC, NS, K, CHUNK)
    dstp = ep[1].reshape(NC, NS, K, CHUNK)

    ones16 = jnp.ones((CHUNK, D_HID), jnp.float32)
    zeros16 = jnp.zeros((ROWS_PER_TILE, D_HID), jnp.float32)
    eye8 = jnp.eye(8, dtype=jnp.float32)
    w1b = W1.astype(jnp.bfloat16)
    w2pad = jnp.pad(W2, ((0, 0), (0, D_HID - D_OUT)))
    w2blk = jnp.kron(eye8, w2pad)
    b1t = jnp.tile(b1, 8)
    b2t = jnp.tile(jnp.pad(b2, (0, D_HID - D_OUT)), 8)
    lane = jnp.arange(128, dtype=jnp.int32)
    ssum = ((lane[:, None] // D_HID == jnp.arange(8)[None, :])
            & (lane[:, None] % D_HID < D_OUT)).astype(jnp.float32)
    sbc = (lane[None, :] // D_HID == jnp.arange(8)[:, None]).astype(
        jnp.float32)

    degp = _deg_kernel(dstp, ones16, zeros16)
    ys2d, dis2d = _tc1(x, x, x, x, x, x, x, x, w1b,
                       degp.reshape(NC, MROWS, 128))
    p1 = _agg_kernel(srcp, dstp, ys2d.reshape(N_PAD, D_HID), zeros16)
    (t2,) = _tc2(p1.reshape(NC, MROWS, 128), dis2d, ys2d, b1t, w2blk)
    p2 = _agg_kernel(srcp, dstp, t2.reshape(N_PAD, D_HID), zeros16)
    (outp,) = _tc3(p2.reshape(NC, MROWS, 128), dis2d, t2, b2t, ssum, sbc)
    outn = outp.reshape(MROWS, 8, D_HID).transpose(1, 0, 2)
    return outn.reshape(N_PAD, D_HID)[:N_NODES, :D_OUT]


# speedup vs baseline: 1.3153x; 1.0741x over previous
import functools

import jax
import jax.numpy as jnp
from jax import lax
from jax.experimental import pallas as pl
from jax.experimental.pallas import tpu as pltpu
from jax.experimental.pallas import tpu_sc as plsc

N_NODES = 10000
N_EDGES = 160000
D_IN = 384
D_HID = 16
D_OUT = 8

NC = 2
NS = 16
CHUNK = 128
K = 40
E_PAD = NC * NS * K * CHUNK
N_PAD = 10240
ROWS_PER_TILE = N_PAD // NS
TROWS = N_PAD // NS

_mesh = plsc.VectorSubcoreMesh(
    core_axis_name="c", subcore_axis_name="s", num_cores=NC, num_subcores=NS)
_sc_params = pltpu.CompilerParams(use_tc_tiling_on_sc=False)


@functools.partial(
    pl.kernel,
    out_type=jax.ShapeDtypeStruct((NC, N_PAD, D_HID), jnp.float32),
    mesh=_mesh,
    scratch_types=[
        pltpu.VMEM((K, CHUNK), jnp.int32),
        pltpu.VMEM((CHUNK, D_HID), jnp.float32),
        pltpu.VMEM_SHARED((N_PAD, D_HID), jnp.float32),
        [pltpu.SemaphoreType.DMA] * 8,
    ],
    compiler_params=_sc_params,
)
def _deg_kernel(dst_hbm, ones_hbm, zeros_hbm, out_hbm, dst_v, ones_v, acc,
                dsems):
    c = lax.axis_index("c")
    s = lax.axis_index("s")
    pltpu.sync_copy(dst_hbm.at[c, s], dst_v)
    pltpu.sync_copy(ones_hbm, ones_v)
    pltpu.sync_copy(zeros_hbm, acc.at[pl.ds(s * ROWS_PER_TILE, ROWS_PER_TILE)])
    plsc.subcore_barrier()

    def body(i, carry):
        cps = [pltpu.async_copy(ones_v, acc.at[dst_v.at[8 * i + u]],
                                dsems[u], add=True) for u in range(8)]
        for cp in cps:
            cp.wait()
        return carry

    lax.fori_loop(0, K // 8, body, 0)
    plsc.subcore_barrier()
    sl = pl.ds(s * ROWS_PER_TILE, ROWS_PER_TILE)
    pltpu.sync_copy(acc.at[sl], out_hbm.at[c].at[sl])


@functools.partial(
    pl.kernel,
    out_type=jax.ShapeDtypeStruct((NC, N_PAD, D_HID), jnp.float32),
    mesh=_mesh,
    scratch_types=[
        pltpu.VMEM((K, CHUNK), jnp.int32),
        pltpu.VMEM((K, CHUNK), jnp.int32),
        pltpu.VMEM((8, CHUNK, D_HID), jnp.float32),
        pltpu.VMEM_SHARED((N_PAD, D_HID), jnp.float32),
        pltpu.VMEM_SHARED((N_PAD, D_HID), jnp.float32),
        [pltpu.SemaphoreType.DMA] * 8,
        [pltpu.SemaphoreType.DMA] * 8,
    ],
    compiler_params=_sc_params,
)
def _agg_kernel(src_hbm, dst_hbm, ys_hbm, zeros_hbm, out_hbm,
                src_v, dst_v, rows_v, table, acc, gsems, ssems):
    c = lax.axis_index("c")
    s = lax.axis_index("s")
    pltpu.sync_copy(src_hbm.at[c, s], src_v)
    pltpu.sync_copy(dst_hbm.at[c, s], dst_v)
    tsl = pl.ds(s * TROWS, TROWS)
    pltpu.sync_copy(ys_hbm.at[tsl], table.at[tsl])
    pltpu.sync_copy(zeros_hbm, acc.at[pl.ds(s * ROWS_PER_TILE, ROWS_PER_TILE)])
    plsc.subcore_barrier()

    def gath(j, u):
        return pltpu.async_copy(table.at[src_v.at[j]], rows_v.at[u], gsems[u])

    def scat(j, u):
        return pltpu.async_copy(rows_v.at[u], acc.at[dst_v.at[j]], ssems[u],
                                add=True)

    def body(i, carry):
        j = 8 * i
        gs = [gath(j + u, u) for u in range(8)]
        ss = []
        for u in range(8):
            gs[u].wait()
            ss.append(scat(j + u, u))
        for s in ss:
            s.wait()
        return carry

    lax.fori_loop(0, K // 8, body, 0)
    plsc.subcore_barrier()
    sl = pl.ds(s * ROWS_PER_TILE, ROWS_PER_TILE)
    pltpu.sync_copy(acc.at[sl], out_hbm.at[c].at[sl])


MROWS = N_PAD * D_HID // 128
_BR = 256
_GRID = MROWS // _BR


def _mm_body(x0, x1, x2, x3, x4, x5, x6, x7, w1_ref, xw_ref):
    w1 = w1_ref[...]
    xw_ref[...] = jnp.concatenate(
        [jnp.dot(xr[...].astype(jnp.bfloat16), w1,
                 preferred_element_type=jnp.float32)
         for xr in (x0, x1, x2, x3, x4, x5, x6, x7)], axis=1)


def _scale_body(xw_ref, degp_ref, ys_ref, dis_ref):
    deg = degp_ref[0] + degp_ref[1] + 1.0
    dis = lax.rsqrt(deg)
    dis_ref[...] = dis
    ys_ref[...] = dis * xw_ref[...]


def _tc2_body(p_ref, dis_ref, ys_ref, b1_ref, w2blk_ref, t2_ref):
    dis = dis_ref[...]
    pre = dis * (p_ref[0] + p_ref[1]) + dis * ys_ref[...] + b1_ref[...]
    h = jnp.maximum(pre, 0.0)
    hw2 = jnp.dot(h, w2blk_ref[...], preferred_element_type=jnp.float32)
    t2_ref[...] = dis * hw2


def _tc3_body(q_ref, dis_ref, t2_ref, b2_ref, ssum_ref, sbc_ref, out_ref):
    dis = dis_ref[...]
    t = dis * (q_ref[0] + q_ref[1]) + dis * t2_ref[...] + b2_ref[...]
    e = jnp.exp(t)
    s8 = jnp.dot(e, ssum_ref[...], preferred_element_type=jnp.float32,
                 precision=lax.Precision.HIGHEST)
    lse = jnp.log(s8)
    lseb = jnp.dot(lse, sbc_ref[...], preferred_element_type=jnp.float32,
                   precision=lax.Precision.HIGHEST)
    out_ref[...] = t - lseb


def _xspec(g):
    return pl.BlockSpec((_BR, D_IN), lambda i, g=g: (_GRID * g + i, 0))


_mm = pl.pallas_call(
    _mm_body,
    grid=(_GRID,),
    in_specs=[_xspec(g) for g in range(8)] + [
        pl.BlockSpec((D_IN, D_HID), lambda i: (0, 0)),
    ],
    out_specs=[pl.BlockSpec((_BR, 128), lambda i: (i, 0))],
    out_shape=[jax.ShapeDtypeStruct((MROWS, 128), jnp.float32)],
)

_scale = pl.pallas_call(
    _scale_body,
    grid=(1,),
    in_specs=[
        pl.BlockSpec((MROWS, 128), lambda i: (0, 0)),
        pl.BlockSpec((NC, MROWS, 128), lambda i: (0, 0, 0)),
    ],
    out_specs=[
        pl.BlockSpec((MROWS, 128), lambda i: (0, 0)),
        pl.BlockSpec((MROWS, 128), lambda i: (0, 0)),
    ],
    out_shape=[
        jax.ShapeDtypeStruct((MROWS, 128), jnp.float32),
        jax.ShapeDtypeStruct((MROWS, 128), jnp.float32),
    ],
)

_tc2 = pl.pallas_call(
    _tc2_body,
    grid=(1,),
    in_specs=[
        pl.BlockSpec((NC, MROWS, 128), lambda i: (0, 0, 0)),
        pl.BlockSpec((MROWS, 128), lambda i: (0, 0)),
        pl.BlockSpec((MROWS, 128), lambda i: (0, 0)),
        pl.BlockSpec((128,), lambda i: (0,)),
        pl.BlockSpec((128, 128), lambda i: (0, 0)),
    ],
    out_specs=[pl.BlockSpec((MROWS, 128), lambda i: (0, 0))],
    out_shape=[jax.ShapeDtypeStruct((MROWS, 128), jnp.float32)],
)

_tc3 = pl.pallas_call(
    _tc3_body,
    grid=(1,),
    in_specs=[
        pl.BlockSpec((NC, MROWS, 128), lambda i: (0, 0, 0)),
        pl.BlockSpec((MROWS, 128), lambda i: (0, 0)),
        pl.BlockSpec((MROWS, 128), lambda i: (0, 0)),
        pl.BlockSpec((128,), lambda i: (0,)),
        pl.BlockSpec((128, 8), lambda i: (0, 0)),
        pl.BlockSpec((8, 128), lambda i: (0, 0)),
    ],
    out_specs=[pl.BlockSpec((MROWS, 128), lambda i: (0, 0))],
    out_shape=[jax.ShapeDtypeStruct((MROWS, 128), jnp.float32)],
)


def kernel(x, edge_index, W1, b1, W2, b2):
    npad = E_PAD - N_EDGES
    q = N_NODES // 8
    em = (edge_index.astype(jnp.int32) % 1280) * 8 \
        + edge_index.astype(jnp.int32) // 1280
    spread = 8 * (1040 + jnp.arange(npad, dtype=jnp.int32) % 240) + 7
    padconst = jnp.concatenate(
        [jnp.zeros((2, N_EDGES), jnp.int32),
         jnp.broadcast_to(spread, (2, npad))], axis=1)
    ep = jnp.pad(em, ((0, 0), (0, npad))) + padconst
    srcp = ep[0].reshape(NC, NS, K, CHUNK)
    dstp = ep[1].reshape(NC, NS, K, CHUNK)

    ones16 = jnp.ones((CHUNK, D_HID), jnp.float32)
    zeros16 = jnp.zeros((ROWS_PER_TILE, D_HID), jnp.float32)
    eye8 = jnp.eye(8, dtype=jnp.float32)
    w1b = W1.astype(jnp.bfloat16)
    w2pad = jnp.pad(W2, ((0, 0), (0, D_HID - D_OUT)))
    w2blk = jnp.kron(eye8, w2pad)
    b1t = jnp.tile(b1, 8)
    b2t = jnp.tile(jnp.pad(b2, (0, D_HID - D_OUT)), 8)
    lane = jnp.arange(128, dtype=jnp.int32)
    ssum = ((lane[:, None] // D_HID == jnp.arange(8)[None, :])
            & (lane[:, None] % D_HID < D_OUT)).astype(jnp.float32)
    sbc = (lane[None, :] // D_HID == jnp.arange(8)[:, None]).astype(
        jnp.float32)

    degp = _deg_kernel(dstp, ones16, zeros16)
    (xw,) = _mm(x, x, x, x, x, x, x, x, w1b)
    ys2d, dis2d = _scale(xw, degp.reshape(NC, MROWS, 128))
    p1 = _agg_kernel(srcp, dstp, ys2d.reshape(N_PAD, D_HID), zeros16)
    (t2,) = _tc2(p1.reshape(NC, MROWS, 128), dis2d, ys2d, b1t, w2blk)
    p2 = _agg_kernel(srcp, dstp, t2.reshape(N_PAD, D_HID), zeros16)
    (outp,) = _tc3(p2.reshape(NC, MROWS, 128), dis2d, t2, b2t, ssum, sbc)
    outn = outp.reshape(MROWS, 8, D_HID).transpose(1, 0, 2)
    return outn.reshape(N_PAD, D_HID)[:N_NODES, :D_OUT]


# speedup vs baseline: 1.3191x; 1.0028x over previous
import functools

import jax
import jax.numpy as jnp
from jax import lax
from jax.experimental import pallas as pl
from jax.experimental.pallas import tpu as pltpu
from jax.experimental.pallas import tpu_sc as plsc

N_NODES = 10000
N_EDGES = 160000
D_IN = 384
D_HID = 16
D_OUT = 8

NC = 2
NS = 16
CHUNK = 128
K = 40
E_PAD = NC * NS * K * CHUNK
N_PAD = 10240
ROWS_PER_TILE = N_PAD // NS
TROWS = N_PAD // NS

_mesh = plsc.VectorSubcoreMesh(
    core_axis_name="c", subcore_axis_name="s", num_cores=NC, num_subcores=NS)
_sc_params = pltpu.CompilerParams(use_tc_tiling_on_sc=False)


@functools.partial(
    pl.kernel,
    out_type=jax.ShapeDtypeStruct((NC, N_PAD, D_HID), jnp.float32),
    mesh=_mesh,
    scratch_types=[
        pltpu.VMEM((K, CHUNK), jnp.int32),
        pltpu.VMEM((CHUNK, D_HID), jnp.float32),
        pltpu.VMEM_SHARED((N_PAD, D_HID), jnp.float32),
        [pltpu.SemaphoreType.DMA] * 4,
    ],
    compiler_params=_sc_params,
)
def _deg_kernel(dst_hbm, ones_hbm, zeros_hbm, out_hbm, dst_v, ones_v, acc,
                dsems):
    c = lax.axis_index("c")
    s = lax.axis_index("s")
    pltpu.sync_copy(dst_hbm.at[c, s], dst_v)
    pltpu.sync_copy(ones_hbm, ones_v)
    pltpu.sync_copy(zeros_hbm, acc.at[pl.ds(s * ROWS_PER_TILE, ROWS_PER_TILE)])
    plsc.subcore_barrier()

    def body(i, carry):
        cps = [pltpu.async_copy(ones_v, acc.at[dst_v.at[4 * i + u]],
                                dsems[u], add=True) for u in range(4)]
        for cp in cps:
            cp.wait()
        return carry

    lax.fori_loop(0, K // 4, body, 0)
    plsc.subcore_barrier()
    sl = pl.ds(s * ROWS_PER_TILE, ROWS_PER_TILE)
    pltpu.sync_copy(acc.at[sl], out_hbm.at[c].at[sl])


@functools.partial(
    pl.kernel,
    out_type=jax.ShapeDtypeStruct((NC, N_PAD, D_HID), jnp.float32),
    mesh=_mesh,
    scratch_types=[
        pltpu.VMEM((K, CHUNK), jnp.int32),
        pltpu.VMEM((K, CHUNK), jnp.int32),
        pltpu.VMEM((8, CHUNK, D_HID), jnp.float32),
        pltpu.VMEM_SHARED((N_PAD, D_HID), jnp.float32),
        pltpu.VMEM_SHARED((N_PAD, D_HID), jnp.float32),
        [pltpu.SemaphoreType.DMA] * 8,
        [pltpu.SemaphoreType.DMA] * 8,
    ],
    compiler_params=_sc_params,
)
def _agg_kernel(src_hbm, dst_hbm, ys_hbm, zeros_hbm, out_hbm,
                src_v, dst_v, rows_v, table, acc, gsems, ssems):
    c = lax.axis_index("c")
    s = lax.axis_index("s")
    pltpu.sync_copy(src_hbm.at[c, s], src_v)
    pltpu.sync_copy(dst_hbm.at[c, s], dst_v)
    tsl = pl.ds(s * TROWS, TROWS)
    pltpu.sync_copy(ys_hbm.at[tsl], table.at[tsl])
    pltpu.sync_copy(zeros_hbm, acc.at[pl.ds(s * ROWS_PER_TILE, ROWS_PER_TILE)])
    plsc.subcore_barrier()

    def gath(j, u):
        return pltpu.async_copy(table.at[src_v.at[j]], rows_v.at[u], gsems[u])

    def scat(j, u):
        return pltpu.async_copy(rows_v.at[u], acc.at[dst_v.at[j]], ssems[u],
                                add=True)

    def body(i, carry):
        j = 8 * i
        gs = [gath(j + u, u) for u in range(8)]
        ss = []
        for u in range(8):
            gs[u].wait()
            ss.append(scat(j + u, u))
        for s in ss:
            s.wait()
        return carry

    lax.fori_loop(0, K // 8, body, 0)
    plsc.subcore_barrier()
    sl = pl.ds(s * ROWS_PER_TILE, ROWS_PER_TILE)
    pltpu.sync_copy(acc.at[sl], out_hbm.at[c].at[sl])


MROWS = N_PAD * D_HID // 128
_BR = 256
_GRID = MROWS // _BR


def _mm_body(x0, x1, x2, x3, x4, x5, x6, x7, w1_ref, xw_ref):
    w1 = w1_ref[...]
    xw_ref[...] = jnp.concatenate(
        [jnp.dot(xr[...].astype(jnp.bfloat16), w1,
                 preferred_element_type=jnp.float32)
         for xr in (x0, x1, x2, x3, x4, x5, x6, x7)], axis=1)


def _scale_body(xw_ref, degp_ref, ys_ref, dis_ref):
    deg = degp_ref[0] + degp_ref[1] + 1.0
    dis = lax.rsqrt(deg)
    dis_ref[...] = dis
    ys_ref[...] = dis * xw_ref[...]


def _tc2_body(p_ref, dis_ref, ys_ref, b1_ref, w2blk_ref, t2_ref):
    dis = dis_ref[...]
    pre = dis * (p_ref[0] + p_ref[1]) + dis * ys_ref[...] + b1_ref[...]
    h = jnp.maximum(pre, 0.0)
    hw2 = jnp.dot(h, w2blk_ref[...], preferred_element_type=jnp.float32)
    t2_ref[...] = dis * hw2


def _tc3_body(q_ref, dis_ref, t2_ref, b2_ref, ssum_ref, sbc_ref, out_ref):
    dis = dis_ref[...]
    t = dis * (q_ref[0] + q_ref[1]) + dis * t2_ref[...] + b2_ref[...]
    e = jnp.exp(t)
    s8 = jnp.dot(e, ssum_ref[...], preferred_element_type=jnp.float32,
                 precision=lax.Precision.HIGHEST)
    lse = jnp.log(s8)
    lseb = jnp.dot(lse, sbc_ref[...], preferred_element_type=jnp.float32,
                   precision=lax.Precision.HIGHEST)
    out_ref[...] = t - lseb


def _xspec(g):
    return pl.BlockSpec((_BR, D_IN), lambda i, g=g: (_GRID * g + i, 0))


_mm = pl.pallas_call(
    _mm_body,
    grid=(_GRID,),
    in_specs=[_xspec(g) for g in range(8)] + [
        pl.BlockSpec((D_IN, D_HID), lambda i: (0, 0)),
    ],
    out_specs=[pl.BlockSpec((_BR, 128), lambda i: (i, 0))],
    out_shape=[jax.ShapeDtypeStruct((MROWS, 128), jnp.float32)],
)

_scale = pl.pallas_call(
    _scale_body,
    grid=(1,),
    in_specs=[
        pl.BlockSpec((MROWS, 128), lambda i: (0, 0)),
        pl.BlockSpec((NC, MROWS, 128), lambda i: (0, 0, 0)),
    ],
    out_specs=[
        pl.BlockSpec((MROWS, 128), lambda i: (0, 0)),
        pl.BlockSpec((MROWS, 128), lambda i: (0, 0)),
    ],
    out_shape=[
        jax.ShapeDtypeStruct((MROWS, 128), jnp.float32),
        jax.ShapeDtypeStruct((MROWS, 128), jnp.float32),
    ],
)

_tc2 = pl.pallas_call(
    _tc2_body,
    grid=(1,),
    in_specs=[
        pl.BlockSpec((NC, MROWS, 128), lambda i: (0, 0, 0)),
        pl.BlockSpec((MROWS, 128), lambda i: (0, 0)),
        pl.BlockSpec((MROWS, 128), lambda i: (0, 0)),
        pl.BlockSpec((128,), lambda i: (0,)),
        pl.BlockSpec((128, 128), lambda i: (0, 0)),
    ],
    out_specs=[pl.BlockSpec((MROWS, 128), lambda i: (0, 0))],
    out_shape=[jax.ShapeDtypeStruct((MROWS, 128), jnp.float32)],
)

_tc3 = pl.pallas_call(
    _tc3_body,
    grid=(1,),
    in_specs=[
        pl.BlockSpec((NC, MROWS, 128), lambda i: (0, 0, 0)),
        pl.BlockSpec((MROWS, 128), lambda i: (0, 0)),
        pl.BlockSpec((MROWS, 128), lambda i: (0, 0)),
        pl.BlockSpec((128,), lambda i: (0,)),
        pl.BlockSpec((128, 8), lambda i: (0, 0)),
        pl.BlockSpec((8, 128), lambda i: (0, 0)),
    ],
    out_specs=[pl.BlockSpec((MROWS, 128), lambda i: (0, 0))],
    out_shape=[jax.ShapeDtypeStruct((MROWS, 128), jnp.float32)],
)


def kernel(x, edge_index, W1, b1, W2, b2):
    npad = E_PAD - N_EDGES
    q = N_NODES // 8
    em = (edge_index.astype(jnp.int32) % 1280) * 8 \
        + edge_index.astype(jnp.int32) // 1280
    spread = 8 * (1040 + jnp.arange(npad, dtype=jnp.int32) % 240) + 7
    padconst = jnp.concatenate(
        [jnp.zeros((2, N_EDGES), jnp.int32),
         jnp.broadcast_to(spread, (2, npad))], axis=1)
    ep = jnp.pad(em, ((0, 0), (0, npad))) + padconst
    srcp = ep[0].reshape(NC, NS, K, CHUNK)
    dstp = ep[1].reshape(NC, NS, K, CHUNK)

    ones16 = jnp.ones((CHUNK, D_HID), jnp.float32)
    zeros16 = jnp.zeros((ROWS_PER_TILE, D_HID), jnp.float32)
    eye8 = jnp.eye(8, dtype=jnp.float32)
    w1b = W1.astype(jnp.bfloat16)
    w2pad = jnp.pad(W2, ((0, 0), (0, D_HID - D_OUT)))
    w2blk = jnp.kron(eye8, w2pad)
    b1t = jnp.tile(b1, 8)
    b2t = jnp.tile(jnp.pad(b2, (0, D_HID - D_OUT)), 8)
    lane = jnp.arange(128, dtype=jnp.int32)
    ssum = ((lane[:, None] // D_HID == jnp.arange(8)[None, :])
            & (lane[:, None] % D_HID < D_OUT)).astype(jnp.float32)
    sbc = (lane[None, :] // D_HID == jnp.arange(8)[:, None]).astype(
        jnp.float32)

    degp = _deg_kernel(dstp, ones16, zeros16)
    (xw,) = _mm(x, x, x, x, x, x, x, x, w1b)
    ys2d, dis2d = _scale(xw, degp.reshape(NC, MROWS, 128))
    p1 = _agg_kernel(srcp, dstp, ys2d.reshape(N_PAD, D_HID), zeros16)
    (t2,) = _tc2(p1.reshape(NC, MROWS, 128), dis2d, ys2d, b1t, w2blk)
    p2 = _agg_kernel(srcp, dstp, t2.reshape(N_PAD, D_HID), zeros16)
    (outp,) = _tc3(p2.reshape(NC, MROWS, 128), dis2d, t2, b2t, ssum, sbc)
    outn = outp.reshape(MROWS, 8, D_HID).transpose(1, 0, 2)
    return outn.reshape(N_PAD, D_HID)[:N_NODES, :D_OUT]


# speedup vs baseline: 1.3901x; 1.0538x over previous
import functools

import jax
import jax.numpy as jnp
from jax import lax
from jax.experimental import pallas as pl
from jax.experimental.pallas import tpu as pltpu
from jax.experimental.pallas import tpu_sc as plsc

N_NODES = 10000
N_EDGES = 160000
D_IN = 384
D_HID = 16
D_OUT = 8

NC = 2
NS = 16
CHUNK = 128
K = 40
E_PAD = NC * NS * K * CHUNK
N_PAD = 10240
ROWS_PER_TILE = N_PAD // NS
TROWS = N_PAD // NS

_mesh = plsc.VectorSubcoreMesh(
    core_axis_name="c", subcore_axis_name="s", num_cores=NC, num_subcores=NS)
_sc_params = pltpu.CompilerParams(use_tc_tiling_on_sc=False)


@functools.partial(
    pl.kernel,
    out_type=jax.ShapeDtypeStruct((NC, N_PAD, D_HID), jnp.float32),
    mesh=_mesh,
    scratch_types=[
        pltpu.VMEM((K, CHUNK), jnp.int32),
        pltpu.VMEM((CHUNK, D_HID), jnp.float32),
        pltpu.VMEM_SHARED((N_PAD, D_HID), jnp.float32),
        [pltpu.SemaphoreType.DMA] * 4,
    ],
    compiler_params=_sc_params,
)
def _deg_kernel(dst_hbm, ones_hbm, zeros_hbm, out_hbm, dst_v, ones_v, acc,
                dsems):
    c = lax.axis_index("c")
    s = lax.axis_index("s")
    c0 = pltpu.async_copy(dst_hbm.at[c, s], dst_v, dsems[0])
    c1 = pltpu.async_copy(ones_hbm, ones_v, dsems[1])
    c2 = pltpu.async_copy(
        zeros_hbm, acc.at[pl.ds(s * ROWS_PER_TILE, ROWS_PER_TILE)], dsems[2])
    c1.wait()
    c2.wait()
    plsc.subcore_barrier()
    c0.wait()

    def body(i, carry):
        cps = [pltpu.async_copy(ones_v, acc.at[dst_v.at[4 * i + u]],
                                dsems[u], add=True) for u in range(4)]
        for cp in cps:
            cp.wait()
        return carry

    lax.fori_loop(0, K // 4, body, 0)
    plsc.subcore_barrier()
    sl = pl.ds(s * ROWS_PER_TILE, ROWS_PER_TILE)
    pltpu.sync_copy(acc.at[sl], out_hbm.at[c].at[sl])


@functools.partial(
    pl.kernel,
    out_type=jax.ShapeDtypeStruct((NC, N_PAD, D_HID), jnp.float32),
    mesh=_mesh,
    scratch_types=[
        pltpu.VMEM((K, CHUNK), jnp.int32),
        pltpu.VMEM((K, CHUNK), jnp.int32),
        pltpu.VMEM((8, CHUNK, D_HID), jnp.float32),
        pltpu.VMEM_SHARED((N_PAD, D_HID), jnp.float32),
        pltpu.VMEM_SHARED((N_PAD, D_HID), jnp.float32),
        [pltpu.SemaphoreType.DMA] * 8,
        [pltpu.SemaphoreType.DMA] * 8,
    ],
    compiler_params=_sc_params,
)
def _agg_kernel(src_hbm, dst_hbm, ys_hbm, zeros_hbm, out_hbm,
                src_v, dst_v, rows_v, table, acc, gsems, ssems):
    c = lax.axis_index("c")
    s = lax.axis_index("s")
    c0 = pltpu.async_copy(src_hbm.at[c, s], src_v, gsems[0])
    c1 = pltpu.async_copy(dst_hbm.at[c, s], dst_v, gsems[1])
    tsl = pl.ds(s * TROWS, TROWS)
    c2 = pltpu.async_copy(ys_hbm.at[tsl], table.at[tsl], gsems[2])
    c3 = pltpu.async_copy(
        zeros_hbm, acc.at[pl.ds(s * ROWS_PER_TILE, ROWS_PER_TILE)], gsems[3])
    c2.wait()
    c3.wait()
    plsc.subcore_barrier()
    c0.wait()
    c1.wait()

    def gath(j, u):
        return pltpu.async_copy(table.at[src_v.at[j]], rows_v.at[u], gsems[u])

    def scat(j, u):
        return pltpu.async_copy(rows_v.at[u], acc.at[dst_v.at[j]], ssems[u],
                                add=True)

    def body(i, carry):
        j = 8 * i
        gs = [gath(j + u, u) for u in range(8)]
        ss = []
        for u in range(8):
            gs[u].wait()
            ss.append(scat(j + u, u))
        for s in ss:
            s.wait()
        return carry

    lax.fori_loop(0, K // 8, body, 0)
    plsc.subcore_barrier()
    sl = pl.ds(s * ROWS_PER_TILE, ROWS_PER_TILE)
    pltpu.sync_copy(acc.at[sl], out_hbm.at[c].at[sl])


MROWS = N_PAD * D_HID // 128
_BR = 256
_GRID = MROWS // _BR


def _mm_body(x0, x1, x2, x3, x4, x5, x6, x7, w1_ref, xw_ref):
    w1 = w1_ref[...]
    xw_ref[...] = jnp.concatenate(
        [jnp.dot(xr[...].astype(jnp.bfloat16), w1,
                 preferred_element_type=jnp.float32)
         for xr in (x0, x1, x2, x3, x4, x5, x6, x7)], axis=1)


def _scale_body(xw_ref, degp_ref, ys_ref, dis_ref):
    deg = degp_ref[0] + degp_ref[1] + 1.0
    dis = lax.rsqrt(deg)
    dis_ref[...] = dis
    ys_ref[...] = dis * xw_ref[...]


def _tc2_body(p_ref, dis_ref, ys_ref, b1_ref, w2blk_ref, t2_ref):
    dis = dis_ref[...]
    pre = dis * (p_ref[0] + p_ref[1]) + dis * ys_ref[...] + b1_ref[...]
    h = jnp.maximum(pre, 0.0)
    hw2 = jnp.dot(h, w2blk_ref[...], preferred_element_type=jnp.float32)
    t2_ref[...] = dis * hw2


def _tc3_body(q_ref, dis_ref, t2_ref, b2_ref, ssum_ref, sbc_ref, out_ref):
    dis = dis_ref[...]
    t = dis * (q_ref[0] + q_ref[1]) + dis * t2_ref[...] + b2_ref[...]
    e = jnp.exp(t)
    s8 = jnp.dot(e, ssum_ref[...], preferred_element_type=jnp.float32,
                 precision=lax.Precision.HIGHEST)
    lse = jnp.log(s8)
    lseb = jnp.dot(lse, sbc_ref[...], preferred_element_type=jnp.float32,
                   precision=lax.Precision.HIGHEST)
    out_ref[...] = t - lseb


def _xspec(g):
    return pl.BlockSpec((_BR, D_IN), lambda i, g=g: (_GRID * g + i, 0))


_mm = pl.pallas_call(
    _mm_body,
    grid=(_GRID,),
    in_specs=[_xspec(g) for g in range(8)] + [
        pl.BlockSpec((D_IN, D_HID), lambda i: (0, 0)),
    ],
    out_specs=[pl.BlockSpec((_BR, 128), lambda i: (i, 0))],
    out_shape=[jax.ShapeDtypeStruct((MROWS, 128), jnp.float32)],
)

_scale = pl.pallas_call(
    _scale_body,
    grid=(1,),
    in_specs=[
        pl.BlockSpec((MROWS, 128), lambda i: (0, 0)),
        pl.BlockSpec((NC, MROWS, 128), lambda i: (0, 0, 0)),
    ],
    out_specs=[
        pl.BlockSpec((MROWS, 128), lambda i: (0, 0)),
        pl.BlockSpec((MROWS, 128), lambda i: (0, 0)),
    ],
    out_shape=[
        jax.ShapeDtypeStruct((MROWS, 128), jnp.float32),
        jax.ShapeDtypeStruct((MROWS, 128), jnp.float32),
    ],
)

_tc2 = pl.pallas_call(
    _tc2_body,
    grid=(1,),
    in_specs=[
        pl.BlockSpec((NC, MROWS, 128), lambda i: (0, 0, 0)),
        pl.BlockSpec((MROWS, 128), lambda i: (0, 0)),
        pl.BlockSpec((MROWS, 128), lambda i: (0, 0)),
        pl.BlockSpec((128,), lambda i: (0,)),
        pl.BlockSpec((128, 128), lambda i: (0, 0)),
    ],
    out_specs=[pl.BlockSpec((MROWS, 128), lambda i: (0, 0))],
    out_shape=[jax.ShapeDtypeStruct((MROWS, 128), jnp.float32)],
)

_tc3 = pl.pallas_call(
    _tc3_body,
    grid=(1,),
    in_specs=[
        pl.BlockSpec((NC, MROWS, 128), lambda i: (0, 0, 0)),
        pl.BlockSpec((MROWS, 128), lambda i: (0, 0)),
        pl.BlockSpec((MROWS, 128), lambda i: (0, 0)),
        pl.BlockSpec((128,), lambda i: (0,)),
        pl.BlockSpec((128, 8), lambda i: (0, 0)),
        pl.BlockSpec((8, 128), lambda i: (0, 0)),
    ],
    out_specs=[pl.BlockSpec((MROWS, 128), lambda i: (0, 0))],
    out_shape=[jax.ShapeDtypeStruct((MROWS, 128), jnp.float32)],
)


def kernel(x, edge_index, W1, b1, W2, b2):
    npad = E_PAD - N_EDGES
    q = N_NODES // 8
    em = (edge_index.astype(jnp.int32) % 1280) * 8 \
        + edge_index.astype(jnp.int32) // 1280
    spread = 8 * (1040 + jnp.arange(npad, dtype=jnp.int32) % 240) + 7
    padconst = jnp.concatenate(
        [jnp.zeros((2, N_EDGES), jnp.int32),
         jnp.broadcast_to(spread, (2, npad))], axis=1)
    ep = jnp.pad(em, ((0, 0), (0, npad))) + padconst
    srcp = ep[0].reshape(NC, NS, K, CHUNK)
    dstp = ep[1].reshape(NC, NS, K, CHUNK)

    ones16 = jnp.ones((CHUNK, D_HID), jnp.float32)
    zeros16 = jnp.zeros((ROWS_PER_TILE, D_HID), jnp.float32)
    eye8 = jnp.eye(8, dtype=jnp.float32)
    w1b = W1.astype(jnp.bfloat16)
    w2pad = jnp.pad(W2, ((0, 0), (0, D_HID - D_OUT)))
    w2blk = jnp.kron(eye8, w2pad)
    b1t = jnp.tile(b1, 8)
    b2t = jnp.tile(jnp.pad(b2, (0, D_HID - D_OUT)), 8)
    lane = jnp.arange(128, dtype=jnp.int32)
    ssum = ((lane[:, None] // D_HID == jnp.arange(8)[None, :])
            & (lane[:, None] % D_HID < D_OUT)).astype(jnp.float32)
    sbc = (lane[None, :] // D_HID == jnp.arange(8)[:, None]).astype(
        jnp.float32)

    degp = _deg_kernel(dstp, ones16, zeros16)
    (xw,) = _mm(x, x, x, x, x, x, x, x, w1b)
    ys2d, dis2d = _scale(xw, degp.reshape(NC, MROWS, 128))
    p1 = _agg_kernel(srcp, dstp, ys2d.reshape(N_PAD, D_HID), zeros16)
    (t2,) = _tc2(p1.reshape(NC, MROWS, 128), dis2d, ys2d, b1t, w2blk)
    p2 = _agg_kernel(srcp, dstp, t2.reshape(N_PAD, D_HID), zeros16)
    (outp,) = _tc3(p2.reshape(NC, MROWS, 128), dis2d, t2, b2t, ssum, sbc)
    outn = outp.reshape(MROWS, 8, D_HID).transpose(1, 0, 2)
    return outn.reshape(N_PAD, D_HID)[:N_NODES, :D_OUT]
